# baseline (speedup 1.0000x reference)
import functools

import jax
import jax.numpy as jnp
from jax import lax
from jax.experimental import pallas as pl
from jax.experimental.pallas import tpu as pltpu
from jax.experimental.pallas import tpu_sc as plsc

TM = 256
GATHER_CH = 48
COMBINE_CT = 8


def _sc_mesh():
    return plsc.VectorSubcoreMesh(core_axis_name="c", subcore_axis_name="s")


def _num_workers():
    info = plsc.get_sparse_core_info()
    return info.num_cores, info.num_subcores, info.num_cores * info.num_subcores


def _make_gather(P, D, nc, nw):
    rows_per_w = P // nw
    n_chunks = rows_per_w // GATHER_CH

    @functools.partial(
        pl.kernel,
        out_type=jax.ShapeDtypeStruct((P, D), jnp.float32),
        mesh=_sc_mesh(),
        scratch_types=[
            pltpu.VMEM((rows_per_w,), jnp.int32),
            pltpu.VMEM((GATHER_CH, D), jnp.float32),
            pltpu.SemaphoreType.DMA,
        ],
    )
    def gather_k(tok_hbm, x_hbm, xs_hbm, idx_v, rows_v, sem):
        wid = lax.axis_index("s") * nc + lax.axis_index("c")
        base = wid * rows_per_w
        pltpu.sync_copy(tok_hbm.at[pl.ds(base, rows_per_w)], idx_v)

        def chunk(ci, carry):
            off = ci * GATHER_CH
            pltpu.async_copy(
                x_hbm.at[idx_v.at[pl.ds(off, GATHER_CH)]], rows_v, sem
            ).wait()
            pltpu.sync_copy(rows_v, xs_hbm.at[pl.ds(base + off, GATHER_CH)])
            return carry

        lax.fori_loop(0, n_chunks, chunk, 0)

    return gather_k


def _make_combine(T, D, P, K, nc, nw):
    toks_per_w = T // nw
    n_chunks = toks_per_w // COMBINE_CT

    @functools.partial(
        pl.kernel,
        out_type=jax.ShapeDtypeStruct((T, D), jnp.float32),
        mesh=_sc_mesh(),
        scratch_types=[
            pltpu.VMEM((toks_per_w,), jnp.int32),
            pltpu.VMEM((toks_per_w,), jnp.int32),
            pltpu.VMEM((COMBINE_CT, D), jnp.float32),
            pltpu.VMEM((COMBINE_CT, D), jnp.float32),
            pltpu.SemaphoreType.DMA,
            pltpu.SemaphoreType.DMA,
        ],
    )
    def combine_k(pa_hbm, pb_hbm, ys_hbm, y_hbm, ia_v, ib_v, ra_v, rb_v, sa, sb):
        wid = lax.axis_index("s") * nc + lax.axis_index("c")
        base = wid * toks_per_w
        pltpu.sync_copy(pa_hbm.at[pl.ds(base, toks_per_w)], ia_v)
        pltpu.sync_copy(pb_hbm.at[pl.ds(base, toks_per_w)], ib_v)

        def chunk(ci, carry):
            off = ci * COMBINE_CT
            cpa = pltpu.async_copy(
                ys_hbm.at[ia_v.at[pl.ds(off, COMBINE_CT)]], ra_v, sa)
            cpb = pltpu.async_copy(
                ys_hbm.at[ib_v.at[pl.ds(off, COMBINE_CT)]], rb_v, sb)
            cpa.wait()
            cpb.wait()

            def rowadd(r, c2):
                for cc in range(D // 16):
                    sl = pl.ds(cc * 16, 16)
                    ra_v[r, sl] = ra_v[r, sl] + rb_v[r, sl]
                return c2

            lax.fori_loop(0, COMBINE_CT, rowadd, 0)
            pltpu.sync_copy(ra_v, y_hbm.at[pl.ds(base + off, COMBINE_CT)])
            return carry

        lax.fori_loop(0, n_chunks, chunk, 0)

    return combine_k


def _tc_swiglu_body(te_ref, xs_ref, rw_ref, g_ref, u_ref, d_ref, o_ref):
    xt = xs_ref[...]
    g = g_ref[0]
    u = u_ref[0]
    dn = d_ref[0]
    a = lax.dot_general(xt, g, (((1,), (1,)), ((), ())),
                        preferred_element_type=jnp.float32)
    b = lax.dot_general(xt, u, (((1,), (1,)), ((), ())),
                        preferred_element_type=jnp.float32)
    h = ((a * jax.nn.sigmoid(a)) * b).astype(jnp.bfloat16)
    o = lax.dot_general(h, dn, (((1,), (1,)), ((), ())),
                        preferred_element_type=jnp.float32)
    o_ref[...] = o * rw_ref[...]


def _make_grouped_swiglu(P, D, FF, E, ntiles):
    grid_spec = pltpu.PrefetchScalarGridSpec(
        num_scalar_prefetch=1,
        grid=(ntiles,),
        in_specs=[
            pl.BlockSpec((TM, D), lambda i, te: (i, 0)),
            pl.BlockSpec((TM, 1), lambda i, te: (i, 0)),
            pl.BlockSpec((1, FF, D), lambda i, te: (te[i], 0, 0)),
            pl.BlockSpec((1, FF, D), lambda i, te: (te[i], 0, 0)),
            pl.BlockSpec((1, D, FF), lambda i, te: (te[i], 0, 0)),
        ],
        out_specs=pl.BlockSpec((TM, D), lambda i, te: (i, 0)),
    )
    return pl.pallas_call(
        _tc_swiglu_body,
        grid_spec=grid_spec,
        out_shape=jax.ShapeDtypeStruct((P, D), jnp.float32),
        compiler_params=pltpu.CompilerParams(
            dimension_semantics=("arbitrary",),
        ),
    )


def kernel(x, token_mask, weights, indices, gate_projs, up_projs, down_projs):
    T, D = x.shape
    E, FF, _ = gate_projs.shape
    K = indices.shape[1]
    TK = T * K
    P = TK + E * TM
    ntiles = P // TM
    nc, _, nw = _num_workers()

    e_flat = indices.reshape(-1).astype(jnp.int32)
    w_flat = (weights * token_mask[:, None].astype(weights.dtype)).reshape(-1)
    oh = (e_flat[:, None] == jnp.arange(E, dtype=jnp.int32)[None, :]).astype(jnp.int32)
    cum = jnp.cumsum(oh, axis=0)
    counts = cum[-1]
    rank = jnp.take_along_axis(cum, e_flat[:, None], axis=1)[:, 0] - 1
    pcounts = ((counts + TM - 1) // TM) * TM
    poff = jnp.concatenate(
        [jnp.zeros((1,), jnp.int32), jnp.cumsum(pcounts)[:-1].astype(jnp.int32)])
    ppos = poff[e_flat] + rank
    tok = jnp.arange(TK, dtype=jnp.int32) // K
    row_token = jnp.zeros((P,), jnp.int32).at[ppos].set(tok)
    row_weight = jnp.zeros((P,), jnp.float32).at[ppos].set(w_flat)
    tile_expert = jnp.clip(
        jnp.searchsorted(poff, jnp.arange(ntiles, dtype=jnp.int32) * TM,
                         side="right") - 1,
        0, E - 1).astype(jnp.int32)
    pos = ppos.reshape(T, K)

    x_pk = lax.bitcast_convert_type(
        x.astype(jnp.bfloat16).reshape(T, D // 2, 2), jnp.float32)
    xs_pk = _make_gather(P, D // 2, nc, nw)(row_token, x_pk)
    xs = lax.bitcast_convert_type(xs_pk, jnp.bfloat16).reshape(P, D)

    ys = _make_grouped_swiglu(P, D, FF, E, ntiles)(
        tile_expert, xs, row_weight.reshape(P, 1),
        gate_projs.astype(jnp.bfloat16), up_projs.astype(jnp.bfloat16),
        down_projs.astype(jnp.bfloat16))

    y = _make_combine(T, D, P, K, nc, nw)(
        pos[:, 0].astype(jnp.int32), pos[:, 1].astype(jnp.int32), ys)
    return y


# speedup vs baseline: 1.8771x; 1.8771x over previous
import functools

import jax
import jax.numpy as jnp
from jax import lax
from jax.experimental import pallas as pl
from jax.experimental.pallas import tpu as pltpu
from jax.experimental.pallas import tpu_sc as plsc

TM = 256
SEGS = 4
GATHER_CH = 24
COMBINE_CT = 8


def _sc_mesh():
    return plsc.VectorSubcoreMesh(core_axis_name="c", subcore_axis_name="s")


def _num_workers():
    info = plsc.get_sparse_core_info()
    return info.num_cores, info.num_subcores, info.num_cores * info.num_subcores


def _make_gather(R, D, nc, nw):
    rows_per_w = R // nw
    ch = GATHER_CH
    n_chunks = rows_per_w // ch
    assert rows_per_w % ch == 0 and n_chunks % 2 == 0

    @functools.partial(
        pl.kernel,
        out_type=jax.ShapeDtypeStruct((R, D), jnp.float32),
        mesh=_sc_mesh(),
        scratch_types=[
            pltpu.VMEM((rows_per_w,), jnp.int32),
            pltpu.VMEM((ch, D), jnp.float32),
            pltpu.VMEM((ch, D), jnp.float32),
            pltpu.SemaphoreType.DMA,
            pltpu.SemaphoreType.DMA,
        ],
    )
    def gather_k(tok_hbm, x_hbm, xs_hbm, idx_v, rows0, rows1, s0, s1):
        wid = lax.axis_index("s") * nc + lax.axis_index("c")
        base = wid * rows_per_w
        pltpu.sync_copy(tok_hbm.at[pl.ds(base, rows_per_w)], idx_v)
        pltpu.async_copy(x_hbm.at[idx_v.at[pl.ds(0, ch)]], rows0, s0)

        def pair(i, carry):
            c1 = 2 * i + 1
            pltpu.async_copy(x_hbm.at[idx_v.at[pl.ds(c1 * ch, ch)]], rows1, s1)
            pltpu.make_async_copy(x_hbm.at[pl.ds(0, ch)], rows0, s0).wait()
            pltpu.sync_copy(rows0, xs_hbm.at[pl.ds(base + 2 * i * ch, ch)])

            @pl.when(2 * i + 2 < n_chunks)
            def _():
                pltpu.async_copy(
                    x_hbm.at[idx_v.at[pl.ds((2 * i + 2) * ch, ch)]], rows0, s0)

            pltpu.make_async_copy(x_hbm.at[pl.ds(0, ch)], rows1, s1).wait()
            pltpu.sync_copy(rows1, xs_hbm.at[pl.ds(base + c1 * ch, ch)])
            return carry

        lax.fori_loop(0, n_chunks // 2, pair, 0)

    return gather_k


def _make_combine(T, D, nc, nw):
    toks_per_w = T // nw
    ct = COMBINE_CT
    n_chunks = toks_per_w // ct
    assert toks_per_w % ct == 0 and n_chunks % 2 == 0

    @functools.partial(
        pl.kernel,
        out_type=jax.ShapeDtypeStruct((T, D), jnp.float32),
        mesh=_sc_mesh(),
        scratch_types=[
            pltpu.VMEM((toks_per_w,), jnp.int32),
            pltpu.VMEM((toks_per_w,), jnp.int32),
            pltpu.VMEM((ct, D), jnp.float32),
            pltpu.VMEM((ct, D), jnp.float32),
            pltpu.VMEM((ct, D), jnp.float32),
            pltpu.VMEM((ct, D), jnp.float32),
            pltpu.SemaphoreType.DMA,
            pltpu.SemaphoreType.DMA,
        ],
    )
    def combine_k(pa_hbm, pb_hbm, ys_hbm, y_hbm,
                  ia_v, ib_v, ra0, rb0, ra1, rb1, s0, s1):
        wid = lax.axis_index("s") * nc + lax.axis_index("c")
        base = wid * toks_per_w
        pltpu.sync_copy(pa_hbm.at[pl.ds(base, toks_per_w)], ia_v)
        pltpu.sync_copy(pb_hbm.at[pl.ds(base, toks_per_w)], ib_v)

        def start(chunk, ra, rb, sem):
            off = chunk * ct
            pltpu.async_copy(ys_hbm.at[ia_v.at[pl.ds(off, ct)]], ra, sem)
            pltpu.async_copy(ys_hbm.at[ib_v.at[pl.ds(off, ct)]], rb, sem)

        def finish(chunk, ra, rb, sem):
            pltpu.make_async_copy(ys_hbm.at[pl.ds(0, ct)], ra, sem).wait()
            pltpu.make_async_copy(ys_hbm.at[pl.ds(0, ct)], rb, sem).wait()

            def rowadd(r, carry):
                for cc in range(D // 16):
                    sl = pl.ds(cc * 16, 16)
                    ra[r, sl] = ra[r, sl] + rb[r, sl]
                return carry

            lax.fori_loop(0, ct, rowadd, 0)
            pltpu.sync_copy(ra, y_hbm.at[pl.ds(base + chunk * ct, ct)])

        start(0, ra0, rb0, s0)

        def pair(i, carry):
            c1 = 2 * i + 1
            start(c1, ra1, rb1, s1)
            finish(2 * i, ra0, rb0, s0)

            @pl.when(2 * i + 2 < n_chunks)
            def _():
                start(2 * i + 2, ra0, rb0, s0)

            finish(c1, ra1, rb1, s1)
            return carry

        lax.fori_loop(0, n_chunks // 2, pair, 0)

    return combine_k


def _tc_swiglu_body(te_ref, xs_ref, rw_ref, g_ref, u_ref, d_ref, ys_any, o_ref):
    xt = xs_ref[...].astype(jnp.bfloat16)
    g = g_ref[0]
    u = u_ref[0]
    dn = d_ref[0]
    a = lax.dot_general(xt, g, (((1,), (1,)), ((), ())),
                        preferred_element_type=jnp.float32)
    b = lax.dot_general(xt, u, (((1,), (1,)), ((), ())),
                        preferred_element_type=jnp.float32)
    h = ((a * jax.nn.sigmoid(a)) * b).astype(jnp.bfloat16)
    o = lax.dot_general(h, dn, (((1,), (1,)), ((), ())),
                        preferred_element_type=jnp.float32)
    o_ref[...] = o * rw_ref[...]


def _make_grouped_swiglu_seg(P, D, FF, seg_tiles, seg_off):
    grid_spec = pltpu.PrefetchScalarGridSpec(
        num_scalar_prefetch=1,
        grid=(seg_tiles,),
        in_specs=[
            pl.BlockSpec((TM, D), lambda i, te: (i, 0)),
            pl.BlockSpec((TM, 1), lambda i, te: (i, 0)),
            pl.BlockSpec((1, FF, D), lambda i, te: (te[i], 0, 0)),
            pl.BlockSpec((1, FF, D), lambda i, te: (te[i], 0, 0)),
            pl.BlockSpec((1, D, FF), lambda i, te: (te[i], 0, 0)),
            pl.BlockSpec(memory_space=pltpu.MemorySpace.HBM),
        ],
        out_specs=pl.BlockSpec((TM, D), lambda i, te: (seg_off + i, 0)),
    )
    return pl.pallas_call(
        _tc_swiglu_body,
        grid_spec=grid_spec,
        out_shape=jax.ShapeDtypeStruct((P, D), jnp.float32),
        input_output_aliases={6: 0},
        compiler_params=pltpu.CompilerParams(
            dimension_semantics=("arbitrary",),
        ),
    )


def kernel(x, token_mask, weights, indices, gate_projs, up_projs, down_projs):
    T, D = x.shape
    E, FF, _ = gate_projs.shape
    K = indices.shape[1]
    TK = T * K
    P = TK + E * TM
    ntiles = P // TM
    nc, _, nw = _num_workers()
    assert ntiles % SEGS == 0
    seg_tiles = ntiles // SEGS
    seg_rows = seg_tiles * TM

    e_flat = indices.reshape(-1).astype(jnp.int32)
    w_flat = (weights * token_mask[:, None].astype(weights.dtype)).reshape(-1)
    oh = (e_flat[:, None] == jnp.arange(E, dtype=jnp.int32)[None, :]).astype(jnp.int32)
    cum = jnp.cumsum(oh, axis=0)
    counts = cum[-1]
    rank = jnp.take_along_axis(cum, e_flat[:, None], axis=1)[:, 0] - 1
    pcounts = ((counts + TM - 1) // TM) * TM
    poff = jnp.concatenate(
        [jnp.zeros((1,), jnp.int32), jnp.cumsum(pcounts)[:-1].astype(jnp.int32)])
    ppos = poff[e_flat] + rank
    tok = jnp.arange(TK, dtype=jnp.int32) // K
    row_token = jnp.zeros((P,), jnp.int32).at[ppos].set(tok)
    row_weight = jnp.zeros((P,), jnp.float32).at[ppos].set(w_flat)
    tile_expert = jnp.clip(
        jnp.searchsorted(poff, jnp.arange(ntiles, dtype=jnp.int32) * TM,
                         side="right") - 1,
        0, E - 1).astype(jnp.int32)
    pos = ppos.reshape(T, K)
    rw2 = row_weight.reshape(P, 1)

    gate_bf = gate_projs.astype(jnp.bfloat16)
    up_bf = up_projs.astype(jnp.bfloat16)
    down_bf = down_projs.astype(jnp.bfloat16)

    gather = _make_gather(seg_rows, D, nc, nw)
    xs_parts = [
        gather(lax.dynamic_slice_in_dim(row_token, s * seg_rows, seg_rows), x)
        for s in range(SEGS)
    ]
    ys = None
    for s in range(SEGS):
        tc = _make_grouped_swiglu_seg(P, D, FF, seg_tiles, s * seg_tiles)
        if ys is None:
            ys = tc(
                lax.dynamic_slice_in_dim(tile_expert, 0, seg_tiles),
                xs_parts[0],
                lax.dynamic_slice_in_dim(rw2, 0, seg_rows),
                gate_bf, up_bf, down_bf,
                jnp.zeros((P, D), jnp.float32))
        else:
            ys = tc(
                lax.dynamic_slice_in_dim(tile_expert, s * seg_tiles, seg_tiles),
                xs_parts[s],
                lax.dynamic_slice_in_dim(rw2, s * seg_rows, seg_rows),
                gate_bf, up_bf, down_bf,
                ys)

    y = _make_combine(T, D, nc, nw)(
        pos[:, 0].astype(jnp.int32), pos[:, 1].astype(jnp.int32), ys)
    return y


# speedup vs baseline: 2.5558x; 1.3615x over previous
import functools

import jax
import jax.numpy as jnp
from jax import lax
from jax.experimental import pallas as pl
from jax.experimental.pallas import tpu as pltpu
from jax.experimental.pallas import tpu_sc as plsc

TM = 256
DISPATCH_CT = 16
COMBINE_CT = 8


def _sc_mesh():
    return plsc.VectorSubcoreMesh(core_axis_name="c", subcore_axis_name="s")


def _num_workers():
    info = plsc.get_sparse_core_info()
    return info.num_cores, info.num_subcores, info.num_cores * info.num_subcores


def _make_dispatch(T, D, P, nc, nw):
    toks_per_w = T // nw
    ct = DISPATCH_CT
    n_chunks = toks_per_w // ct
    assert toks_per_w % ct == 0 and n_chunks % 2 == 0

    @functools.partial(
        pl.kernel,
        out_type=(jax.ShapeDtypeStruct((P, D), jnp.float32),
                  jax.ShapeDtypeStruct((P,), jnp.float32)),
        mesh=_sc_mesh(),
        scratch_types=[
            pltpu.VMEM((n_chunks, ct), jnp.int32),
            pltpu.VMEM((n_chunks, ct), jnp.int32),
            pltpu.VMEM((n_chunks, ct), jnp.float32),
            pltpu.VMEM((n_chunks, ct), jnp.float32),
            pltpu.VMEM((ct, D), jnp.float32),
            pltpu.VMEM((ct, D), jnp.float32),
            pltpu.SemaphoreType.DMA,
            pltpu.SemaphoreType.DMA,
        ],
    )
    def dispatch_k(pa_hbm, pb_hbm, wa_hbm, wb_hbm, x_hbm, xs_hbm, rw_hbm,
                   ia_v, ib_v, wa_v, wb_v, r0, r1, s0, s1):
        wid = lax.axis_index("s") * nc + lax.axis_index("c")
        base = wid * toks_per_w
        rowbase = wid * n_chunks
        pltpu.sync_copy(pa_hbm.at[pl.ds(rowbase, n_chunks)], ia_v)
        pltpu.sync_copy(pb_hbm.at[pl.ds(rowbase, n_chunks)], ib_v)
        pltpu.sync_copy(wa_hbm.at[pl.ds(rowbase, n_chunks)], wa_v)
        pltpu.sync_copy(wb_hbm.at[pl.ds(rowbase, n_chunks)], wb_v)

        def load(chunk, buf):
            pltpu.sync_copy(x_hbm.at[pl.ds(base + chunk * ct, ct)], buf)

        def scat(chunk, buf, sem):
            pltpu.async_copy(buf, xs_hbm.at[ia_v.at[chunk]], sem)
            pltpu.async_copy(buf, xs_hbm.at[ib_v.at[chunk]], sem)
            pltpu.async_copy(wa_v.at[chunk], rw_hbm.at[ia_v.at[chunk]], sem)
            pltpu.async_copy(wb_v.at[chunk], rw_hbm.at[ib_v.at[chunk]], sem)

        def drain(buf, sem):
            pltpu.make_async_copy(buf, xs_hbm.at[pl.ds(0, ct)], sem).wait()
            pltpu.make_async_copy(buf, xs_hbm.at[pl.ds(0, ct)], sem).wait()
            pltpu.make_async_copy(wa_v.at[0], rw_hbm.at[pl.ds(0, ct)], sem).wait()
            pltpu.make_async_copy(wb_v.at[0], rw_hbm.at[pl.ds(0, ct)], sem).wait()

        load(0, r0)
        scat(0, r0, s0)

        def pair(i, carry):
            c1 = 2 * i + 1
            load(c1, r1)
            scat(c1, r1, s1)
            drain(r0, s0)

            @pl.when(2 * i + 2 < n_chunks)
            def _():
                load(2 * i + 2, r0)
                scat(2 * i + 2, r0, s0)

            drain(r1, s1)
            return carry

        lax.fori_loop(0, n_chunks // 2, pair, 0)

    return dispatch_k


def _make_combine(T, D, nc, nw):
    toks_per_w = T // nw
    ct = COMBINE_CT
    n_chunks = toks_per_w // ct
    assert toks_per_w % ct == 0 and n_chunks % 2 == 0

    @functools.partial(
        pl.kernel,
        out_type=jax.ShapeDtypeStruct((T, D), jnp.float32),
        mesh=_sc_mesh(),
        scratch_types=[
            pltpu.VMEM((toks_per_w,), jnp.int32),
            pltpu.VMEM((toks_per_w,), jnp.int32),
            pltpu.VMEM((ct, D), jnp.float32),
            pltpu.VMEM((ct, D), jnp.float32),
            pltpu.VMEM((ct, D), jnp.float32),
            pltpu.VMEM((ct, D), jnp.float32),
            pltpu.SemaphoreType.DMA,
            pltpu.SemaphoreType.DMA,
        ],
    )
    def combine_k(pa_hbm, pb_hbm, ys_hbm, y_hbm,
                  ia_v, ib_v, ra0, rb0, ra1, rb1, s0, s1):
        wid = lax.axis_index("s") * nc + lax.axis_index("c")
        base = wid * toks_per_w
        pltpu.sync_copy(pa_hbm.at[pl.ds(base, toks_per_w)], ia_v)
        pltpu.sync_copy(pb_hbm.at[pl.ds(base, toks_per_w)], ib_v)

        def start(chunk, ra, rb, sem):
            off = chunk * ct
            pltpu.async_copy(ys_hbm.at[ia_v.at[pl.ds(off, ct)]], ra, sem)
            pltpu.async_copy(ys_hbm.at[ib_v.at[pl.ds(off, ct)]], rb, sem)

        def finish(chunk, ra, rb, sem):
            pltpu.make_async_copy(ys_hbm.at[pl.ds(0, ct)], ra, sem).wait()
            pltpu.make_async_copy(ys_hbm.at[pl.ds(0, ct)], rb, sem).wait()
            off = chunk * ct

            def rowcomb(r, carry):
                for cc in range(D // 16):
                    sl = pl.ds(cc * 16, 16)
                    ra[r, sl] = ra[r, sl] + rb[r, sl]
                return carry

            lax.fori_loop(0, ct, rowcomb, 0)
            pltpu.sync_copy(ra, y_hbm.at[pl.ds(base + off, ct)])

        start(0, ra0, rb0, s0)

        def pair(i, carry):
            c1 = 2 * i + 1
            start(c1, ra1, rb1, s1)
            finish(2 * i, ra0, rb0, s0)

            @pl.when(2 * i + 2 < n_chunks)
            def _():
                start(2 * i + 2, ra0, rb0, s0)

            finish(c1, ra1, rb1, s1)
            return carry

        lax.fori_loop(0, n_chunks // 2, pair, 0)

    return combine_k


def _tc_swiglu_body(te_ref, xs_ref, rw_ref, g_ref, u_ref, d_ref, o_ref):
    xt = xs_ref[...].astype(jnp.bfloat16)
    g = g_ref[0]
    u = u_ref[0]
    dn = d_ref[0]
    a = lax.dot_general(xt, g, (((1,), (1,)), ((), ())),
                        preferred_element_type=jnp.float32)
    b = lax.dot_general(xt, u, (((1,), (1,)), ((), ())),
                        preferred_element_type=jnp.float32)
    h = ((a * jax.nn.sigmoid(a)) * b).astype(jnp.bfloat16)
    o = lax.dot_general(h, dn, (((1,), (1,)), ((), ())),
                        preferred_element_type=jnp.float32)
    o_ref[...] = o * rw_ref[...]


def _make_grouped_swiglu(P, D, FF, ntiles):
    grid_spec = pltpu.PrefetchScalarGridSpec(
        num_scalar_prefetch=1,
        grid=(ntiles,),
        in_specs=[
            pl.BlockSpec((TM, D), lambda i, te: (i, 0)),
            pl.BlockSpec((TM, 1), lambda i, te: (i, 0)),
            pl.BlockSpec((1, FF, D), lambda i, te: (te[i], 0, 0)),
            pl.BlockSpec((1, FF, D), lambda i, te: (te[i], 0, 0)),
            pl.BlockSpec((1, D, FF), lambda i, te: (te[i], 0, 0)),
        ],
        out_specs=pl.BlockSpec((TM, D), lambda i, te: (i, 0)),
    )
    return pl.pallas_call(
        _tc_swiglu_body,
        grid_spec=grid_spec,
        out_shape=jax.ShapeDtypeStruct((P, D), jnp.float32),
        compiler_params=pltpu.CompilerParams(
            dimension_semantics=("arbitrary",),
        ),
    )


def kernel(x, token_mask, weights, indices, gate_projs, up_projs, down_projs):
    T, D = x.shape
    E, FF, _ = gate_projs.shape
    K = indices.shape[1]
    TK = T * K
    P = TK + E * TM
    ntiles = P // TM
    nc, _, nw = _num_workers()

    e_flat = indices.reshape(-1).astype(jnp.int32)
    w_flat = (weights * token_mask[:, None].astype(weights.dtype)).reshape(-1)
    oh = (e_flat[:, None] == jnp.arange(E, dtype=jnp.int32)[None, :]).astype(jnp.int32)
    cum = jnp.cumsum(oh, axis=0)
    counts = cum[-1]
    rank = jnp.take_along_axis(cum, e_flat[:, None], axis=1)[:, 0] - 1
    pcounts = ((counts + TM - 1) // TM) * TM
    poff = jnp.concatenate(
        [jnp.zeros((1,), jnp.int32), jnp.cumsum(pcounts)[:-1].astype(jnp.int32)])
    ppos = poff[e_flat] + rank
    tile_expert = jnp.clip(
        jnp.searchsorted(poff, jnp.arange(ntiles, dtype=jnp.int32) * TM,
                         side="right") - 1,
        0, E - 1).astype(jnp.int32)
    pos = ppos.reshape(T, K)
    pa = pos[:, 0].astype(jnp.int32)
    pb = pos[:, 1].astype(jnp.int32)
    wk = w_flat.reshape(T, K)

    ct = DISPATCH_CT
    xs, rw = _make_dispatch(T, D, P, nc, nw)(
        pa.reshape(T // ct, ct), pb.reshape(T // ct, ct),
        wk[:, 0].reshape(T // ct, ct), wk[:, 1].reshape(T // ct, ct),
        x)

    ys = _make_grouped_swiglu(P, D, FF, ntiles)(
        tile_expert, xs, rw.reshape(P, 1),
        gate_projs.astype(jnp.bfloat16), up_projs.astype(jnp.bfloat16),
        down_projs.astype(jnp.bfloat16))

    y = _make_combine(T, D, nc, nw)(pa, pb, ys)
    return y


# speedup vs baseline: 2.5920x; 1.0142x over previous
import functools

import jax
import jax.numpy as jnp
from jax import lax
from jax.experimental import pallas as pl
from jax.experimental.pallas import tpu as pltpu
from jax.experimental.pallas import tpu_sc as plsc

TM = 256
DISPATCH_CT = 16
COMBINE_CT = 8


def _sc_mesh():
    return plsc.VectorSubcoreMesh(core_axis_name="c", subcore_axis_name="s")


def _num_workers():
    info = plsc.get_sparse_core_info()
    return info.num_cores, info.num_subcores, info.num_cores * info.num_subcores


def _make_dispatch(T, D, P, nc, nw):
    toks_per_w = T // nw
    ct = DISPATCH_CT
    n_chunks = toks_per_w // ct
    assert toks_per_w % ct == 0 and n_chunks % 2 == 0

    wct = 128
    n_wch = toks_per_w // wct
    assert toks_per_w % wct == 0

    @functools.partial(
        pl.kernel,
        out_type=(jax.ShapeDtypeStruct((P, D), jnp.float32),
                  jax.ShapeDtypeStruct((P,), jnp.float32)),
        mesh=_sc_mesh(),
        scratch_types=[
            pltpu.VMEM((n_chunks, ct), jnp.int32),
            pltpu.VMEM((n_chunks, ct), jnp.int32),
            pltpu.VMEM((n_wch, wct), jnp.int32),
            pltpu.VMEM((n_wch, wct), jnp.int32),
            pltpu.VMEM((n_wch, wct), jnp.float32),
            pltpu.VMEM((n_wch, wct), jnp.float32),
            pltpu.VMEM((ct, D), jnp.float32),
            pltpu.VMEM((ct, D), jnp.float32),
            pltpu.SemaphoreType.DMA,
            pltpu.SemaphoreType.DMA,
            pltpu.SemaphoreType.DMA,
        ],
    )
    def dispatch_k(pa_hbm, pb_hbm, pa128_hbm, pb128_hbm, wa_hbm, wb_hbm,
                   x_hbm, xs_hbm, rw_hbm,
                   ia_v, ib_v, ja_v, jb_v, wa_v, wb_v, r0, r1, s0, s1, sw):
        wid = lax.axis_index("s") * nc + lax.axis_index("c")
        base = wid * toks_per_w
        rowbase = wid * n_chunks
        pltpu.sync_copy(pa_hbm.at[pl.ds(rowbase, n_chunks)], ia_v)
        pltpu.sync_copy(pb_hbm.at[pl.ds(rowbase, n_chunks)], ib_v)
        wrow = wid * n_wch
        pltpu.sync_copy(pa128_hbm.at[pl.ds(wrow, n_wch)], ja_v)
        pltpu.sync_copy(pb128_hbm.at[pl.ds(wrow, n_wch)], jb_v)
        pltpu.sync_copy(wa_hbm.at[pl.ds(wrow, n_wch)], wa_v)
        pltpu.sync_copy(wb_hbm.at[pl.ds(wrow, n_wch)], wb_v)
        for j in range(n_wch):
            pltpu.async_copy(wa_v.at[j], rw_hbm.at[ja_v.at[j]], sw)
            pltpu.async_copy(wb_v.at[j], rw_hbm.at[jb_v.at[j]], sw)

        def load(chunk, buf):
            pltpu.sync_copy(x_hbm.at[pl.ds(base + chunk * ct, ct)], buf)

        def scat(chunk, buf, sem):
            pltpu.async_copy(buf, xs_hbm.at[ia_v.at[chunk]], sem)
            pltpu.async_copy(buf, xs_hbm.at[ib_v.at[chunk]], sem)

        def drain(buf, sem):
            pltpu.make_async_copy(buf, xs_hbm.at[pl.ds(0, ct)], sem).wait()
            pltpu.make_async_copy(buf, xs_hbm.at[pl.ds(0, ct)], sem).wait()

        load(0, r0)
        scat(0, r0, s0)

        def pair(i, carry):
            c1 = 2 * i + 1
            load(c1, r1)
            scat(c1, r1, s1)
            drain(r0, s0)

            @pl.when(2 * i + 2 < n_chunks)
            def _():
                load(2 * i + 2, r0)
                scat(2 * i + 2, r0, s0)

            drain(r1, s1)
            return carry

        lax.fori_loop(0, n_chunks // 2, pair, 0)
        for j in range(2 * n_wch):
            pltpu.make_async_copy(wa_v.at[0], rw_hbm.at[pl.ds(0, wct)], sw).wait()

    return dispatch_k


def _make_combine(T, D, nc, nw):
    toks_per_w = T // nw
    ct = COMBINE_CT
    n_chunks = toks_per_w // ct
    assert toks_per_w % ct == 0 and n_chunks % 2 == 0

    @functools.partial(
        pl.kernel,
        out_type=jax.ShapeDtypeStruct((T, D), jnp.float32),
        mesh=_sc_mesh(),
        scratch_types=[
            pltpu.VMEM((toks_per_w,), jnp.int32),
            pltpu.VMEM((toks_per_w,), jnp.int32),
            pltpu.VMEM((ct, D), jnp.float32),
            pltpu.VMEM((ct, D), jnp.float32),
            pltpu.VMEM((ct, D), jnp.float32),
            pltpu.VMEM((ct, D), jnp.float32),
            pltpu.SemaphoreType.DMA,
            pltpu.SemaphoreType.DMA,
        ],
    )
    def combine_k(pa_hbm, pb_hbm, ys_hbm, y_hbm,
                  ia_v, ib_v, ra0, rb0, ra1, rb1, s0, s1):
        wid = lax.axis_index("s") * nc + lax.axis_index("c")
        base = wid * toks_per_w
        pltpu.sync_copy(pa_hbm.at[pl.ds(base, toks_per_w)], ia_v)
        pltpu.sync_copy(pb_hbm.at[pl.ds(base, toks_per_w)], ib_v)

        def start(chunk, ra, rb, sem):
            off = chunk * ct
            pltpu.async_copy(ys_hbm.at[ia_v.at[pl.ds(off, ct)]], ra, sem)
            pltpu.async_copy(ys_hbm.at[ib_v.at[pl.ds(off, ct)]], rb, sem)

        def finish(chunk, ra, rb, sem):
            pltpu.make_async_copy(ys_hbm.at[pl.ds(0, ct)], ra, sem).wait()
            pltpu.make_async_copy(ys_hbm.at[pl.ds(0, ct)], rb, sem).wait()
            off = chunk * ct

            def rowcomb(r, carry):
                for cc in range(D // 16):
                    sl = pl.ds(cc * 16, 16)
                    ra[r, sl] = ra[r, sl] + rb[r, sl]
                return carry

            lax.fori_loop(0, ct, rowcomb, 0)
            pltpu.sync_copy(ra, y_hbm.at[pl.ds(base + off, ct)])

        start(0, ra0, rb0, s0)

        def pair(i, carry):
            c1 = 2 * i + 1
            start(c1, ra1, rb1, s1)
            finish(2 * i, ra0, rb0, s0)

            @pl.when(2 * i + 2 < n_chunks)
            def _():
                start(2 * i + 2, ra0, rb0, s0)

            finish(c1, ra1, rb1, s1)
            return carry

        lax.fori_loop(0, n_chunks // 2, pair, 0)

    return combine_k


def _tc_swiglu_body(te_ref, xs_ref, rw_ref, g_ref, u_ref, d_ref, o_ref):
    xt = xs_ref[...].astype(jnp.bfloat16)
    g = g_ref[0]
    u = u_ref[0]
    dn = d_ref[0]
    a = lax.dot_general(xt, g, (((1,), (1,)), ((), ())),
                        preferred_element_type=jnp.float32)
    b = lax.dot_general(xt, u, (((1,), (1,)), ((), ())),
                        preferred_element_type=jnp.float32)
    h = ((a * jax.nn.sigmoid(a)) * b).astype(jnp.bfloat16)
    o = lax.dot_general(h, dn, (((1,), (1,)), ((), ())),
                        preferred_element_type=jnp.float32)
    o_ref[...] = o * rw_ref[...]


def _make_grouped_swiglu(P, D, FF, ntiles):
    grid_spec = pltpu.PrefetchScalarGridSpec(
        num_scalar_prefetch=1,
        grid=(ntiles,),
        in_specs=[
            pl.BlockSpec((TM, D), lambda i, te: (i, 0)),
            pl.BlockSpec((TM, 1), lambda i, te: (i, 0)),
            pl.BlockSpec((1, FF, D), lambda i, te: (te[i], 0, 0)),
            pl.BlockSpec((1, FF, D), lambda i, te: (te[i], 0, 0)),
            pl.BlockSpec((1, D, FF), lambda i, te: (te[i], 0, 0)),
        ],
        out_specs=pl.BlockSpec((TM, D), lambda i, te: (i, 0)),
    )
    return pl.pallas_call(
        _tc_swiglu_body,
        grid_spec=grid_spec,
        out_shape=jax.ShapeDtypeStruct((P, D), jnp.float32),
        compiler_params=pltpu.CompilerParams(
            dimension_semantics=("arbitrary",),
        ),
    )


def kernel(x, token_mask, weights, indices, gate_projs, up_projs, down_projs):
    T, D = x.shape
    E, FF, _ = gate_projs.shape
    K = indices.shape[1]
    TK = T * K
    P = TK + E * TM
    ntiles = P // TM
    nc, _, nw = _num_workers()

    e_flat = indices.T.reshape(-1).astype(jnp.int32)
    w_flat = (weights * token_mask[:, None].astype(weights.dtype)).T.reshape(-1)
    oh = (e_flat[:, None] == jnp.arange(E, dtype=jnp.int32)[None, :]).astype(jnp.int32)
    cum = jnp.cumsum(oh, axis=0)
    counts = cum[-1]
    rank = jnp.take_along_axis(cum, e_flat[:, None], axis=1)[:, 0] - 1
    pcounts = ((counts + TM - 1) // TM) * TM
    poff = jnp.concatenate(
        [jnp.zeros((1,), jnp.int32), jnp.cumsum(pcounts)[:-1].astype(jnp.int32)])
    ppos = poff[e_flat] + rank
    ends = (poff + pcounts).astype(jnp.int32)
    tile_starts = jnp.arange(ntiles, dtype=jnp.int32) * TM
    tile_expert = jnp.minimum(
        jnp.sum((tile_starts[:, None] >= ends[None, :]).astype(jnp.int32),
                axis=1),
        E - 1).astype(jnp.int32)
    pa = ppos[:T]
    pb = ppos[T:]
    wa = w_flat[:T]
    wb = w_flat[T:]

    ct = DISPATCH_CT
    xs, rw = _make_dispatch(T, D, P, nc, nw)(
        pa.reshape(T // ct, ct), pb.reshape(T // ct, ct),
        pa.reshape(T // 128, 128), pb.reshape(T // 128, 128),
        wa.reshape(T // 128, 128), wb.reshape(T // 128, 128),
        x)

    ys = _make_grouped_swiglu(P, D, FF, ntiles)(
        tile_expert, xs, rw.reshape(P, 1),
        gate_projs.astype(jnp.bfloat16), up_projs.astype(jnp.bfloat16),
        down_projs.astype(jnp.bfloat16))

    y = _make_combine(T, D, nc, nw)(pa, pb, ys)
    return y
